# baseline (speedup 1.0000x reference)
import functools

import jax
import jax.numpy as jnp
from jax import lax
from jax.experimental import pallas as pl
from jax.experimental.pallas import tpu as pltpu
from jax.experimental.pallas import tpu_sc as plsc

N = 10000
E = 320000
R = 8
D = 128

NC = 2
NS = 16
NW = NC * NS

NPAD = 10240
APAD = NPAD * R
ST_LEN = N * 16
ST_PAD = 160256
E_PAD = NW * NPAD
EROWS = E_PAD // 128
RPW = EROWS // NW
RPC = 16
EC = RPC * 128
NCHUNK = RPW // RPC

_HI = jax.lax.Precision.HIGHEST


def _tc_pre_body(h_ref, w_ref, aw_ref, ab_ref, st_ref, hs_ref, w16_ref):
    i = pl.program_id(0)

    @pl.when(i == 0)
    def _():
        a1 = aw_ref[0:D, :]
        a2 = aw_ref[D:2 * D, :]
        cols = []
        for r in range(R):
            cols.append(jnp.dot(w_ref[r], a1, precision=_HI,
                                preferred_element_type=jnp.float32))
        for r in range(R):
            cols.append(jnp.dot(w_ref[r], a2, precision=_HI,
                                preferred_element_type=jnp.float32))
        w16_ref[...] = jnp.concatenate(cols, axis=1)

    h_blk = h_ref[...]
    st = jnp.dot(h_blk, w16_ref[...], precision=_HI,
                 preferred_element_type=jnp.float32)
    lane = lax.broadcasted_iota(jnp.int32, (1, 16), 1)
    st = st + jnp.where(lane < 8, ab_ref[0, 0], 0.0)
    st_ref[...] = st

    part = jnp.sum(h_blk, axis=0, keepdims=True)

    @pl.when(i == 0)
    def _():
        hs_ref[...] = part

    @pl.when(i > 0)
    def _():
        hs_ref[...] = hs_ref[...] + part


def _tc_pre(h_node, weight, a_W, a_b):
    nb = 10
    blk = N // nb
    return pl.pallas_call(
        _tc_pre_body,
        grid=(nb,),
        in_specs=[
            pl.BlockSpec((blk, D), lambda i: (i, 0)),
            pl.BlockSpec((R, D, D), lambda i: (0, 0, 0)),
            pl.BlockSpec((2 * D, 1), lambda i: (0, 0)),
            pl.BlockSpec((1, 1), lambda i: (0, 0)),
        ],
        out_specs=[
            pl.BlockSpec((blk, 16), lambda i: (i, 0)),
            pl.BlockSpec((1, D), lambda i: (0, 0)),
        ],
        out_shape=[
            jax.ShapeDtypeStruct((N, 16), jnp.float32),
            jax.ShapeDtypeStruct((1, D), jnp.float32),
        ],
        scratch_shapes=[pltpu.VMEM((D, 16), jnp.float32)],
    )(h_node, weight, a_W, a_b)


def _sc1_body(st_hbm, src_hbm, dst_hbm, rel_hbm,
              den_out, ex_out,
              st_sp, den_sp,
              src_c, dst_c, rel_c, i1_c, i2_c, g1_c, g2_c, ex_c, zb, stage,
              sem):
    cid = lax.axis_index("c")
    sid = lax.axis_index("s")
    wid = sid * NC + cid

    stw = ST_PAD // NS
    pltpu.sync_copy(st_hbm.at[pl.ds(sid * stw, stw)], stage)
    pltpu.sync_copy(stage, st_sp.at[pl.ds(sid * stw, stw)])

    def _zb(j, _):
        zb[pl.ds(j * 16, 16)] = jnp.zeros((16,), jnp.float32)
        return 0
    lax.fori_loop(0, (NPAD // NS) // 16, _zb, 0)
    pltpu.sync_copy(zb, den_sp.at[pl.ds(sid * (NPAD // NS), NPAD // NS)])
    plsc.subcore_barrier()

    base = wid * RPW * 128

    def chunk(i, _):
        e0 = base + i * EC
        pltpu.sync_copy(src_hbm.at[pl.ds(e0, EC)], src_c)
        pltpu.sync_copy(dst_hbm.at[pl.ds(e0, EC)], dst_c)
        pltpu.sync_copy(rel_hbm.at[pl.ds(e0, EC)], rel_c)

        def vec1(j, _):
            sl = pl.ds(j * 16, 16)
            i1_c[sl] = src_c[sl] * 16 + rel_c[sl]
            i2_c[sl] = dst_c[sl] * 16 + (rel_c[sl] + 8)
            return 0
        lax.fori_loop(0, EC // 16, vec1, 0)

        d1 = pltpu.make_async_copy(st_sp.at[i1_c], g1_c, sem)
        d2 = pltpu.make_async_copy(st_sp.at[i2_c], g2_c, sem)
        d1.start()
        d2.start()
        d1.wait()
        d2.wait()

        def vec2(j, _):
            sl = pl.ds(j * 16, 16)
            s = g1_c[sl] + g2_c[sl]
            s = jnp.maximum(s, 0.2 * s)
            ex_c[sl] = jnp.exp(s)
            return 0
        lax.fori_loop(0, EC // 16, vec2, 0)

        pltpu.sync_copy(ex_c, ex_out.at[pl.ds(e0, EC)])
        pltpu.sync_copy(ex_c, den_sp.at[dst_c], add=True)
        return 0

    lax.fori_loop(0, NCHUNK, chunk, 0)

    plsc.subcore_barrier()
    dw = NPAD // NS
    pltpu.sync_copy(den_sp.at[pl.ds(sid * dw, dw)], zb)
    pltpu.sync_copy(zb, den_out.at[cid, pl.ds(sid * dw, dw)])


def _sc_phase1(st_flat, src2, dst2, rel2):
    mesh = plsc.VectorSubcoreMesh(core_axis_name="c", subcore_axis_name="s",
                                  num_cores=NC, num_subcores=NS)
    f = functools.partial(
        pl.kernel,
        out_type=(
            jax.ShapeDtypeStruct((NC, NPAD), jnp.float32),
            jax.ShapeDtypeStruct((E_PAD,), jnp.float32),
        ),
        mesh=mesh,
        scratch_types=[
            pltpu.VMEM_SHARED((ST_PAD,), jnp.float32),
            pltpu.VMEM_SHARED((NPAD,), jnp.float32),
            pltpu.VMEM((EC,), jnp.int32),
            pltpu.VMEM((EC,), jnp.int32),
            pltpu.VMEM((EC,), jnp.int32),
            pltpu.VMEM((EC,), jnp.int32),
            pltpu.VMEM((EC,), jnp.int32),
            pltpu.VMEM((EC,), jnp.float32),
            pltpu.VMEM((EC,), jnp.float32),
            pltpu.VMEM((EC,), jnp.float32),
            pltpu.VMEM((NPAD // NS,), jnp.float32),
            pltpu.VMEM((ST_PAD // NS,), jnp.float32),
            pltpu.SemaphoreType.DMA,
        ],
    )(_sc1_body)
    return f(st_flat, src2, dst2, rel2)


def _sc2_body(ex_hbm, src_hbm, dst_hbm, rel_hbm, den_hbm,
              a_out,
              a_sp, den_sp,
              ex_c, src_c, dst_c, rel_c, dg_c, al_c, ia_c, db0, db1, zb, sem):
    cid = lax.axis_index("c")
    sid = lax.axis_index("s")
    wid = sid * NC + cid

    aw = APAD // NS

    def _zb(j, _):
        zb[pl.ds(j * 16, 16)] = jnp.zeros((16,), jnp.float32)
        return 0
    lax.fori_loop(0, aw // 16, _zb, 0)
    pltpu.sync_copy(zb, a_sp.at[pl.ds(sid * aw, aw)])

    dw = NPAD // NS
    pltpu.sync_copy(den_hbm.at[0, pl.ds(sid * dw, dw)], db0)
    pltpu.sync_copy(den_hbm.at[1, pl.ds(sid * dw, dw)], db1)

    def _dsum(j, _):
        db0[pl.ds(j * 16, 16)] = (db0[pl.ds(j * 16, 16)]
                                  + db1[pl.ds(j * 16, 16)])
        return 0
    lax.fori_loop(0, dw // 16, _dsum, 0)
    pltpu.sync_copy(db0, den_sp.at[pl.ds(sid * dw, dw)])
    plsc.subcore_barrier()

    base = wid * RPW * 128

    def chunk(i, _):
        e0 = base + i * EC
        pltpu.sync_copy(ex_hbm.at[pl.ds(e0, EC)], ex_c)
        pltpu.sync_copy(src_hbm.at[pl.ds(e0, EC)], src_c)
        pltpu.sync_copy(dst_hbm.at[pl.ds(e0, EC)], dst_c)
        pltpu.sync_copy(rel_hbm.at[pl.ds(e0, EC)], rel_c)

        dg = pltpu.make_async_copy(den_sp.at[dst_c], dg_c, sem)
        dg.start()
        dg.wait()

        def vec(j, _):
            sl = pl.ds(j * 16, 16)
            den = jnp.maximum(dg_c[sl], 1e-12)
            al_c[sl] = ex_c[sl] / den
            ia_c[sl] = src_c[sl] * 8 + rel_c[sl]
            return 0
        lax.fori_loop(0, EC // 16, vec, 0)

        pltpu.sync_copy(al_c, a_sp.at[ia_c], add=True)
        return 0

    lax.fori_loop(0, NCHUNK, chunk, 0)

    plsc.subcore_barrier()
    pltpu.sync_copy(a_sp.at[pl.ds(sid * aw, aw)], zb)
    pltpu.sync_copy(zb, a_out.at[cid, pl.ds(sid * aw, aw)])


def _sc_phase2(ex2, src2, dst2, rel2, den2):
    mesh = plsc.VectorSubcoreMesh(core_axis_name="c", subcore_axis_name="s",
                                  num_cores=NC, num_subcores=NS)
    f = functools.partial(
        pl.kernel,
        out_type=jax.ShapeDtypeStruct((NC, APAD), jnp.float32),
        mesh=mesh,
        scratch_types=[
            pltpu.VMEM_SHARED((APAD,), jnp.float32),
            pltpu.VMEM_SHARED((NPAD,), jnp.float32),
            pltpu.VMEM((EC,), jnp.float32),
            pltpu.VMEM((EC,), jnp.int32),
            pltpu.VMEM((EC,), jnp.int32),
            pltpu.VMEM((EC,), jnp.int32),
            pltpu.VMEM((EC,), jnp.float32),
            pltpu.VMEM((EC,), jnp.float32),
            pltpu.VMEM((EC,), jnp.int32),
            pltpu.VMEM((NPAD // NS,), jnp.float32),
            pltpu.VMEM((NPAD // NS,), jnp.float32),
            pltpu.VMEM((APAD // NS,), jnp.float32),
            pltpu.SemaphoreType.DMA,
        ],
    )(_sc2_body)
    return f(ex2, src2, dst2, rel2, den2)


def _tc_post_body(a_ref, h_ref, w_ref, lw_ref, hs_ref, u_ref, gw_ref, gb_ref,
                  out_ref, v_ref):
    i = pl.program_id(0)
    nb = pl.num_programs(0)

    a_blk = a_ref[0] + a_ref[1]
    h_blk = h_ref[...]
    prod = lax.dot_general(a_blk, h_blk, (((0,), (0,)), ((), ())),
                           precision=_HI,
                           preferred_element_type=jnp.float32)

    @pl.when(i == 0)
    def _():
        v_ref[...] = prod

    @pl.when(i > 0)
    def _():
        v_ref[...] = v_ref[...] + prod

    @pl.when(i == nb - 1)
    def _():
        v = v_ref[...]
        s_agg = jnp.zeros((1, D), jnp.float32)
        for r in range(R):
            s_agg = s_agg + jnp.dot(v[r:r + 1, :], w_ref[r], precision=_HI,
                                    preferred_element_type=jnp.float32)
        s_loop = jnp.dot(hs_ref[...], lw_ref[...], precision=_HI,
                         preferred_element_type=jnp.float32)
        sum_h = s_agg + s_loop
        z = (jnp.dot(u_ref[...], gw_ref[0:D, :], precision=_HI,
                     preferred_element_type=jnp.float32)
             + jnp.dot(sum_h, gw_ref[D:2 * D, :], precision=_HI,
                       preferred_element_type=jnp.float32)
             + gb_ref[...])
        out_ref[...] = jnp.maximum(z, 0.0)


def _tc_post(a2, h_node, weight, loop_weight, h_sum, u, gW, gb):
    nb = 10
    blk = N // nb
    return pl.pallas_call(
        _tc_post_body,
        grid=(nb,),
        in_specs=[
            pl.BlockSpec((2, blk, 8), lambda i: (0, i, 0)),
            pl.BlockSpec((blk, D), lambda i: (i, 0)),
            pl.BlockSpec((R, D, D), lambda i: (0, 0, 0)),
            pl.BlockSpec((D, D), lambda i: (0, 0)),
            pl.BlockSpec((1, D), lambda i: (0, 0)),
            pl.BlockSpec((1, D), lambda i: (0, 0)),
            pl.BlockSpec((2 * D, D), lambda i: (0, 0)),
            pl.BlockSpec((1, D), lambda i: (0, 0)),
        ],
        out_specs=pl.BlockSpec((1, D), lambda i: (0, 0)),
        out_shape=jax.ShapeDtypeStruct((1, D), jnp.float32),
        scratch_shapes=[pltpu.VMEM((R, D), jnp.float32)],
    )(a2, h_node, weight, loop_weight, h_sum, u, gW, gb)


def kernel(h_node, edge_index, rel_type, u, weight, loop_weight,
           apply_node_W, apply_node_b, apply_global_W, apply_global_b,
           a_W, a_b):
    del apply_node_W, apply_node_b
    src = edge_index[0]
    dst = edge_index[1]

    pe = E_PAD - E
    src2 = jnp.concatenate([src, jnp.full((pe,), N, jnp.int32)])
    dst2 = jnp.concatenate([dst, jnp.full((pe,), N, jnp.int32)])
    rel2 = jnp.concatenate([rel_type, jnp.zeros((pe,), jnp.int32)])

    st, h_sum = _tc_pre(h_node, weight, a_W, a_b.reshape(1, 1))
    st_flat = jnp.concatenate(
        [st.reshape(-1), jnp.full((ST_PAD - ST_LEN,), -1e9, jnp.float32)])

    den2, ex2 = _sc_phase1(st_flat, src2, dst2, rel2)
    a2 = _sc_phase2(ex2, src2, dst2, rel2, den2)
    a2r = a2[:, :N * R].reshape(NC, N, R)

    return _tc_post(a2r, h_node, weight, loop_weight, h_sum, u,
                    apply_global_W, apply_global_b.reshape(1, D))


# speedup vs baseline: 1.2809x; 1.2809x over previous
import functools

import jax
import jax.numpy as jnp
from jax import lax
from jax.experimental import pallas as pl
from jax.experimental.pallas import tpu as pltpu
from jax.experimental.pallas import tpu_sc as plsc

N = 10000
E = 320000
R = 8
D = 128

NC = 2
NS = 16
NW = NC * NS

NPAD = 10240
APAD = NPAD * R
ST_LEN = N * 16
EPW = E // NW
EC = 2000
NCHUNK = EPW // EC
EROWS = E // 128

_HI = jax.lax.Precision.HIGHEST


def _tc_pre_body(h_ref, w_ref, aw_ref, ab_ref, s_ref, d_ref, r_ref,
                 st_ref, hs_ref, i1_ref, i2_ref, ia_ref, w16_ref):
    i = pl.program_id(0)

    @pl.when(i == 0)
    def _():
        a1 = aw_ref[0:D, :]
        a2 = aw_ref[D:2 * D, :]
        cols = []
        for r in range(R):
            cols.append(jnp.dot(w_ref[r], a1, precision=_HI,
                                preferred_element_type=jnp.float32))
        for r in range(R):
            cols.append(jnp.dot(w_ref[r], a2, precision=_HI,
                                preferred_element_type=jnp.float32))
        w16_ref[...] = jnp.concatenate(cols, axis=1)

    h_blk = h_ref[...]
    st = jnp.dot(h_blk, w16_ref[...], precision=_HI,
                 preferred_element_type=jnp.float32)
    lane = lax.broadcasted_iota(jnp.int32, (1, 16), 1)
    st_ref[...] = st + jnp.where(lane < 8, ab_ref[0, 0], 0.0)

    @pl.when(i == 0)
    def _():
        sv = s_ref[...]
        dv = d_ref[...]
        rv = r_ref[...]
        i1_ref[...] = sv * 16 + rv
        i2_ref[...] = dv * 16 + (rv + 8)
        ia_ref[...] = sv * 8 + rv

    part = jnp.sum(h_blk, axis=0, keepdims=True)

    @pl.when(i == 0)
    def _():
        hs_ref[...] = part

    @pl.when(i > 0)
    def _():
        hs_ref[...] = hs_ref[...] + part


def _tc_pre(h_node, weight, a_W, a_b, src2, dst2, rel2):
    nb = 10
    blk = N // nb
    return pl.pallas_call(
        _tc_pre_body,
        grid=(nb,),
        in_specs=[
            pl.BlockSpec((blk, D), lambda i: (i, 0)),
            pl.BlockSpec((R, D, D), lambda i: (0, 0, 0)),
            pl.BlockSpec((2 * D, 1), lambda i: (0, 0)),
            pl.BlockSpec((1, 1), lambda i: (0, 0)),
            pl.BlockSpec((EROWS, 128), lambda i: (0, 0)),
            pl.BlockSpec((EROWS, 128), lambda i: (0, 0)),
            pl.BlockSpec((EROWS, 128), lambda i: (0, 0)),
        ],
        out_specs=[
            pl.BlockSpec((blk, 16), lambda i: (i, 0)),
            pl.BlockSpec((1, D), lambda i: (0, 0)),
            pl.BlockSpec((EROWS, 128), lambda i: (0, 0)),
            pl.BlockSpec((EROWS, 128), lambda i: (0, 0)),
            pl.BlockSpec((EROWS, 128), lambda i: (0, 0)),
        ],
        out_shape=[
            jax.ShapeDtypeStruct((N, 16), jnp.float32),
            jax.ShapeDtypeStruct((1, D), jnp.float32),
            jax.ShapeDtypeStruct((EROWS, 128), jnp.int32),
            jax.ShapeDtypeStruct((EROWS, 128), jnp.int32),
            jax.ShapeDtypeStruct((EROWS, 128), jnp.int32),
        ],
        scratch_shapes=[pltpu.VMEM((D, 16), jnp.float32)],
    )(h_node, weight, a_W, a_b, src2, dst2, rel2)


def _sc1_body(st_hbm, i1_hbm, i2_hbm, dst_hbm,
              den_out, ex_out,
              st_sp, den_sp,
              i1b0, i1b1, i2b0, i2b1, dstb0, dstb1, g1b, g2b, exb0, exb1,
              zb, stage,
              sin0, sin1, sg, so0, so1):
    cid = lax.axis_index("c")
    sid = lax.axis_index("s")
    wid = sid * NC + cid

    sin = [sin0, sin1]
    so = [so0, so1]
    i1b = [i1b0, i1b1]
    i2b = [i2b0, i2b1]
    dstb = [dstb0, dstb1]
    exb = [exb0, exb1]

    stw = ST_LEN // NS
    pltpu.sync_copy(st_hbm.at[pl.ds(sid * stw, stw)], stage)
    pltpu.sync_copy(stage, st_sp.at[pl.ds(sid * stw, stw)])

    def _zb(j, _):
        zb[pl.ds(j * 16, 16)] = jnp.zeros((16,), jnp.float32)
        return 0
    lax.fori_loop(0, (NPAD // NS) // 16, _zb, 0)
    pltpu.sync_copy(zb, den_sp.at[pl.ds(sid * (NPAD // NS), NPAD // NS)])
    plsc.subcore_barrier()

    base = wid * EPW

    def start_in(i, s):
        e0 = base + i * EC
        descs = [
            pltpu.make_async_copy(i1_hbm.at[pl.ds(e0, EC)], i1b[s], sin[s]),
            pltpu.make_async_copy(i2_hbm.at[pl.ds(e0, EC)], i2b[s], sin[s]),
            pltpu.make_async_copy(dst_hbm.at[pl.ds(e0, EC)], dstb[s], sin[s]),
        ]
        for d in descs:
            d.start()
        return descs

    pend = [None, None]
    pend[0] = start_in(0, 0)
    for i in range(NCHUNK):
        s = i % 2
        if i + 1 < NCHUNK:
            pend[1 - s] = start_in(i + 1, 1 - s)
        for d in pend[s]:
            d.wait()

        d1 = pltpu.make_async_copy(st_sp.at[i1b[s]], g1b, sg)
        d2 = pltpu.make_async_copy(st_sp.at[i2b[s]], g2b, sg)
        d1.start()
        d2.start()
        d1.wait()
        d2.wait()

        exs = exb[s]

        def vec(j, _):
            sl = pl.ds(j * 16, 16)
            v = g1b[sl] + g2b[sl]
            v = jnp.maximum(v, 0.2 * v)
            exs[sl] = jnp.exp(v)
            return 0
        lax.fori_loop(0, EC // 16, vec, 0)

        e0 = base + i * EC
        pltpu.sync_copy(exb[s], ex_out.at[pl.ds(e0, EC)])
        pltpu.sync_copy(exb[s], den_sp.at[dstb[s]], add=True)

    plsc.subcore_barrier()
    dw = NPAD // NS
    pltpu.sync_copy(den_sp.at[pl.ds(sid * dw, dw)], zb)
    pltpu.sync_copy(zb, den_out.at[cid, pl.ds(sid * dw, dw)])


def _sc_phase1(st_flat, i1, i2, dst):
    mesh = plsc.VectorSubcoreMesh(core_axis_name="c", subcore_axis_name="s",
                                  num_cores=NC, num_subcores=NS)
    f = functools.partial(
        pl.kernel,
        out_type=(
            jax.ShapeDtypeStruct((NC, NPAD), jnp.float32),
            jax.ShapeDtypeStruct((E,), jnp.float32),
        ),
        mesh=mesh,
        scratch_types=[
            pltpu.VMEM_SHARED((ST_LEN,), jnp.float32),
            pltpu.VMEM_SHARED((NPAD,), jnp.float32),
            pltpu.VMEM((EC,), jnp.int32),
            pltpu.VMEM((EC,), jnp.int32),
            pltpu.VMEM((EC,), jnp.int32),
            pltpu.VMEM((EC,), jnp.int32),
            pltpu.VMEM((EC,), jnp.int32),
            pltpu.VMEM((EC,), jnp.int32),
            pltpu.VMEM((EC,), jnp.float32),
            pltpu.VMEM((EC,), jnp.float32),
            pltpu.VMEM((EC,), jnp.float32),
            pltpu.VMEM((EC,), jnp.float32),
            pltpu.VMEM((NPAD // NS,), jnp.float32),
            pltpu.VMEM((ST_LEN // NS,), jnp.float32),
            pltpu.SemaphoreType.DMA,
            pltpu.SemaphoreType.DMA,
            pltpu.SemaphoreType.DMA,
            pltpu.SemaphoreType.DMA,
            pltpu.SemaphoreType.DMA,
        ],
    )(_sc1_body)
    return f(st_flat, i1, i2, dst)


def _sc2_body(ex_hbm, dst_hbm, ia_hbm, den_hbm,
              a_out,
              a_sp, den_sp,
              exb0, exb1, dstb0, dstb1, iab0, iab1, dgb, alb0, alb1,
              db0, db1, zb,
              sin0, sin1, sg, so0, so1):
    cid = lax.axis_index("c")
    sid = lax.axis_index("s")
    wid = sid * NC + cid

    sin = [sin0, sin1]
    so = [so0, so1]
    exb = [exb0, exb1]
    dstb = [dstb0, dstb1]
    iab = [iab0, iab1]
    alb = [alb0, alb1]

    aw = APAD // NS

    def _zb(j, _):
        zb[pl.ds(j * 16, 16)] = jnp.zeros((16,), jnp.float32)
        return 0
    lax.fori_loop(0, aw // 16, _zb, 0)
    pltpu.sync_copy(zb, a_sp.at[pl.ds(sid * aw, aw)])

    dw = NPAD // NS
    pltpu.sync_copy(den_hbm.at[0, pl.ds(sid * dw, dw)], db0)
    pltpu.sync_copy(den_hbm.at[1, pl.ds(sid * dw, dw)], db1)

    def _dsum(j, _):
        db0[pl.ds(j * 16, 16)] = (db0[pl.ds(j * 16, 16)]
                                  + db1[pl.ds(j * 16, 16)])
        return 0
    lax.fori_loop(0, dw // 16, _dsum, 0)
    pltpu.sync_copy(db0, den_sp.at[pl.ds(sid * dw, dw)])
    plsc.subcore_barrier()

    base = wid * EPW

    def start_in(i, s):
        e0 = base + i * EC
        descs = [
            pltpu.make_async_copy(ex_hbm.at[pl.ds(e0, EC)], exb[s], sin[s]),
            pltpu.make_async_copy(dst_hbm.at[pl.ds(e0, EC)], dstb[s], sin[s]),
            pltpu.make_async_copy(ia_hbm.at[pl.ds(e0, EC)], iab[s], sin[s]),
        ]
        for d in descs:
            d.start()
        return descs

    pend = [None, None]
    pend[0] = start_in(0, 0)
    for i in range(NCHUNK):
        s = i % 2
        if i + 1 < NCHUNK:
            pend[1 - s] = start_in(i + 1, 1 - s)
        for d in pend[s]:
            d.wait()

        dg = pltpu.make_async_copy(den_sp.at[dstb[s]], dgb, sg)
        dg.start()
        dg.wait()
        als = alb[s]
        exs = exb[s]

        def vec(j, _):
            sl = pl.ds(j * 16, 16)
            den = jnp.maximum(dgb[sl], 1e-12)
            als[sl] = exs[sl] / den
            return 0
        lax.fori_loop(0, EC // 16, vec, 0)

        pltpu.sync_copy(alb[s], a_sp.at[iab[s]], add=True)

    plsc.subcore_barrier()
    pltpu.sync_copy(a_sp.at[pl.ds(sid * aw, aw)], zb)
    pltpu.sync_copy(zb, a_out.at[cid, pl.ds(sid * aw, aw)])


def _sc_phase2(ex, dst, ia, den2):
    mesh = plsc.VectorSubcoreMesh(core_axis_name="c", subcore_axis_name="s",
                                  num_cores=NC, num_subcores=NS)
    f = functools.partial(
        pl.kernel,
        out_type=jax.ShapeDtypeStruct((NC, APAD), jnp.float32),
        mesh=mesh,
        scratch_types=[
            pltpu.VMEM_SHARED((APAD,), jnp.float32),
            pltpu.VMEM_SHARED((NPAD,), jnp.float32),
            pltpu.VMEM((EC,), jnp.float32),
            pltpu.VMEM((EC,), jnp.float32),
            pltpu.VMEM((EC,), jnp.int32),
            pltpu.VMEM((EC,), jnp.int32),
            pltpu.VMEM((EC,), jnp.int32),
            pltpu.VMEM((EC,), jnp.int32),
            pltpu.VMEM((EC,), jnp.float32),
            pltpu.VMEM((EC,), jnp.float32),
            pltpu.VMEM((EC,), jnp.float32),
            pltpu.VMEM((NPAD // NS,), jnp.float32),
            pltpu.VMEM((NPAD // NS,), jnp.float32),
            pltpu.VMEM((APAD // NS,), jnp.float32),
            pltpu.SemaphoreType.DMA,
            pltpu.SemaphoreType.DMA,
            pltpu.SemaphoreType.DMA,
            pltpu.SemaphoreType.DMA,
            pltpu.SemaphoreType.DMA,
        ],
    )(_sc2_body)
    return f(ex, dst, ia, den2)


def _tc_post_body(a_ref, h_ref, w_ref, lw_ref, hs_ref, u_ref, gw_ref, gb_ref,
                  out_ref, v_ref):
    i = pl.program_id(0)
    nb = pl.num_programs(0)

    a_blk = a_ref[0] + a_ref[1]
    h_blk = h_ref[...]
    prod = lax.dot_general(a_blk, h_blk, (((0,), (0,)), ((), ())),
                           precision=_HI,
                           preferred_element_type=jnp.float32)

    @pl.when(i == 0)
    def _():
        v_ref[...] = prod

    @pl.when(i > 0)
    def _():
        v_ref[...] = v_ref[...] + prod

    @pl.when(i == nb - 1)
    def _():
        v = v_ref[...]
        s_agg = jnp.zeros((1, D), jnp.float32)
        for r in range(R):
            s_agg = s_agg + jnp.dot(v[r:r + 1, :], w_ref[r], precision=_HI,
                                    preferred_element_type=jnp.float32)
        s_loop = jnp.dot(hs_ref[...], lw_ref[...], precision=_HI,
                         preferred_element_type=jnp.float32)
        sum_h = s_agg + s_loop
        z = (jnp.dot(u_ref[...], gw_ref[0:D, :], precision=_HI,
                     preferred_element_type=jnp.float32)
             + jnp.dot(sum_h, gw_ref[D:2 * D, :], precision=_HI,
                       preferred_element_type=jnp.float32)
             + gb_ref[...])
        out_ref[...] = jnp.maximum(z, 0.0)


def _tc_post(a2, h_node, weight, loop_weight, h_sum, u, gW, gb):
    nb = 10
    blk = N // nb
    return pl.pallas_call(
        _tc_post_body,
        grid=(nb,),
        in_specs=[
            pl.BlockSpec((2, blk, 8), lambda i: (0, i, 0)),
            pl.BlockSpec((blk, D), lambda i: (i, 0)),
            pl.BlockSpec((R, D, D), lambda i: (0, 0, 0)),
            pl.BlockSpec((D, D), lambda i: (0, 0)),
            pl.BlockSpec((1, D), lambda i: (0, 0)),
            pl.BlockSpec((1, D), lambda i: (0, 0)),
            pl.BlockSpec((2 * D, D), lambda i: (0, 0)),
            pl.BlockSpec((1, D), lambda i: (0, 0)),
        ],
        out_specs=pl.BlockSpec((1, D), lambda i: (0, 0)),
        out_shape=jax.ShapeDtypeStruct((1, D), jnp.float32),
        scratch_shapes=[pltpu.VMEM((R, D), jnp.float32)],
    )(a2, h_node, weight, loop_weight, h_sum, u, gW, gb)


def kernel(h_node, edge_index, rel_type, u, weight, loop_weight,
           apply_node_W, apply_node_b, apply_global_W, apply_global_b,
           a_W, a_b):
    del apply_node_W, apply_node_b
    src = edge_index[0]
    dst = edge_index[1]

    src2 = src.reshape(EROWS, 128)
    dst2 = dst.reshape(EROWS, 128)
    rel2 = rel_type.reshape(EROWS, 128)

    st, h_sum, i1o, i2o, iao = _tc_pre(h_node, weight, a_W,
                                       a_b.reshape(1, 1), src2, dst2, rel2)

    den2, ex = _sc_phase1(st.reshape(-1), i1o.reshape(-1), i2o.reshape(-1),
                          dst)
    a2 = _sc_phase2(ex, dst, iao.reshape(-1), den2)
    a2r = a2[:, :N * R].reshape(NC, N, R)

    return _tc_post(a2r, h_node, weight, loop_weight, h_sum, u,
                    apply_global_W, apply_global_b.reshape(1, D))
